# baseline (speedup 1.0000x reference)
import functools

import numpy as np

import jax
import jax.numpy as jnp
from jax import lax
from jax.experimental import pallas as pl
from jax.experimental.pallas import tpu as pltpu
from jax.experimental.pallas import tpu_sc as plsc

_NEG = np.float32(-3.0e38)


def _make_sc_counts(m, n, num_segments):
    mesh = plsc.VectorSubcoreMesh(core_axis_name="c", subcore_axis_name="s")

    @functools.partial(
        pl.kernel,
        out_type=jax.ShapeDtypeStruct((2, num_segments, n), jnp.float32),
        mesh=mesh,
        compiler_params=pltpu.CompilerParams(needs_layout_passes=False),
        scratch_types=[
            pltpu.VMEM((m,), jnp.int32),
            pltpu.VMEM((m + 32,), jnp.int32),
            pltpu.VMEM((n,), jnp.float32),
        ],
    )
    def sc_kernel(idx_hbm, seg_hbm, c_out, seg_v, idx_v, crow_v):
        c = lax.axis_index("c")
        t = lax.axis_index("s")
        pltpu.sync_copy(seg_hbm, seg_v)
        pltpu.sync_copy(idx_hbm, idx_v.at[pl.ds(0, m)])
        lanes = lax.iota(jnp.int32, 16)
        one = jnp.float32(1.0)
        nil = jnp.float32(0.0)
        zf16 = jnp.zeros((16,), jnp.float32)
        ones16 = jnp.full((16,), 1.0, jnp.float32)

        def cz_body(k, carry):
            s_acc, e_acc = carry
            v = seg_v[pl.ds(k * 16, 16)]
            crow_v[pl.ds(k * 16, 16)] = zf16
            s_acc = s_acc + jnp.where(v < t, one, nil)
            e_acc = e_acc + jnp.where(v <= t, one, nil)
            return s_acc, e_acc

        assert m == n and m % 16 == 0
        s_acc, e_acc = lax.fori_loop(0, m // 16, cz_body, (zf16, zf16), unroll=8)
        start = jnp.sum(s_acc).astype(jnp.int32)
        end = jnp.sum(e_acc).astype(jnp.int32)

        mid = (start + end) // 2
        h0 = jnp.where(c == 0, start, mid)
        h1 = jnp.where(c == 0, mid, end)
        nch = (h1 - h0 + 31) // 32

        def sc_body(i, carry):
            pos = h0 + i * 32
            valid0 = (lanes + pos) < h1
            valid1 = (lanes + (pos + 16)) < h1
            iv0 = idx_v[pl.ds(pos, 16)]
            iv1 = idx_v[pl.ds(pos + 16, 16)]
            plsc.addupdate_scatter(crow_v, [iv0], ones16, mask=valid0)
            plsc.addupdate_scatter(crow_v, [iv1], ones16, mask=valid1)
            return carry

        lax.fori_loop(0, nch, sc_body, 0)
        pltpu.sync_copy(crow_v, c_out.at[c, t])

    return sc_kernel


def _mv_body(z_ref, w_ref, c2_ref, y_ref, md_ref, m_run, d_run):
    k = pl.program_id(0)
    nseg = c2_ref.shape[1]
    blk = c2_ref.shape[2]
    y_blk = jnp.sum(z_ref[...] * w_ref[...], axis=1)
    y_ref[...] = y_blk[None, None, :]

    @pl.when(k == 0)
    def _():
        m_run[...] = jnp.full((nseg, 1), _NEG, jnp.float32)
        d_run[...] = jnp.zeros((nseg, 1), jnp.float32)

    cb = c2_ref[0] + c2_ref[1]
    yb = jnp.broadcast_to(y_blk[None, :], (nseg, blk))
    ymasked = jnp.where(cb > 0.0, yb, _NEG)
    bmax = jnp.max(ymasked, axis=1, keepdims=True)
    m_new = jnp.maximum(m_run[...], bmax)
    e_blk = jnp.where(cb > 0.0, jnp.exp(yb - m_new), 0.0)
    d_new = d_run[...] * jnp.exp(m_run[...] - m_new) + jnp.sum(
        cb * e_blk, axis=1, keepdims=True
    )
    m_run[...] = m_new
    d_run[...] = d_new

    @pl.when(k == pl.num_programs(0) - 1)
    def _():
        m_fin = jnp.where(m_new == _NEG, 0.0, m_new)
        d_fin = jnp.where(d_new == 0.0, 1.0, d_new)
        md_ref[:, 0:16] = jnp.broadcast_to(m_fin, (nseg, 16))
        md_ref[:, 16:32] = jnp.broadcast_to(d_fin, (nseg, 16))


def _matvec_stats(z, w, c2, num_segments):
    n, dim = z.shape
    blk = 2048
    grid = n // blk
    return pl.pallas_call(
        _mv_body,
        grid=(grid,),
        in_specs=[
            pl.BlockSpec((blk, dim), lambda k: (k, 0)),
            pl.BlockSpec((1, dim), lambda k: (0, 0)),
            pl.BlockSpec((2, num_segments, blk), lambda k: (0, 0, k)),
        ],
        out_specs=[
            pl.BlockSpec((1, 1, blk), lambda k: (k, 0, 0)),
            pl.BlockSpec((num_segments, 32), lambda k: (0, 0)),
        ],
        out_shape=[
            jax.ShapeDtypeStruct((grid, 1, blk), jnp.float32),
            jax.ShapeDtypeStruct((num_segments, 32), jnp.float32),
        ],
        scratch_shapes=[
            pltpu.VMEM((num_segments, 1), jnp.float32),
            pltpu.VMEM((num_segments, 1), jnp.float32),
        ],
    )(z, w.reshape(1, dim), c2)


def _mm_body(c2_ref, md_ref, y_ref, z_ref, out_ref):
    k = pl.program_id(0)
    nseg = c2_ref.shape[1]
    blk = c2_ref.shape[2]
    mvec = md_ref[:, 0:1]
    inv_d = 1.0 / md_ref[:, 16:17]
    cb = c2_ref[0] + c2_ref[1]
    yb = jnp.broadcast_to(y_ref[0], (nseg, blk))
    s_blk = jnp.where(cb > 0.0, cb * jnp.exp(yb - mvec) * inv_d, 0.0)
    part = jnp.dot(s_blk, z_ref[...], preferred_element_type=jnp.float32)

    @pl.when(k == 0)
    def _():
        out_ref[...] = part

    @pl.when(k > 0)
    def _():
        out_ref[...] += part


def _weighted_matmul(c2, md, y3d, z, num_segments):
    n, dim = z.shape
    grid, _, blk = y3d.shape
    return pl.pallas_call(
        _mm_body,
        grid=(grid,),
        in_specs=[
            pl.BlockSpec((2, num_segments, blk), lambda k: (0, 0, k)),
            pl.BlockSpec((num_segments, 32), lambda k: (0, 0)),
            pl.BlockSpec((1, 1, blk), lambda k: (k, 0, 0)),
            pl.BlockSpec((blk, dim), lambda k: (k, 0)),
        ],
        out_specs=pl.BlockSpec((num_segments, dim), lambda k: (0, 0)),
        out_shape=jax.ShapeDtypeStruct((num_segments, dim), jnp.float32),
    )(c2, md, y3d, z)


def kernel(z, w_attn, b_attn, flat_idx, segment_ids):
    del b_attn
    n, dim = z.shape
    (m,) = flat_idx.shape
    num_segments = 16
    idx32 = flat_idx.astype(jnp.int32)
    seg32 = segment_ids.astype(jnp.int32)
    c2 = _make_sc_counts(m, n, num_segments)(idx32, seg32)
    y3d, md = _matvec_stats(z, w_attn, c2, num_segments)
    return _weighted_matmul(c2, md, y3d, z, num_segments)


# speedup vs baseline: 1.3918x; 1.3918x over previous
import functools

import numpy as np

import jax
import jax.numpy as jnp
from jax import lax
from jax.experimental import pallas as pl
from jax.experimental.pallas import tpu as pltpu
from jax.experimental.pallas import tpu_sc as plsc

_NEG = np.float32(-3.0e38)


def _make_sc_counts(m, n, num_segments):
    mesh = plsc.VectorSubcoreMesh(core_axis_name="c", subcore_axis_name="s")

    @functools.partial(
        pl.kernel,
        out_type=jax.ShapeDtypeStruct((2, num_segments, n), jnp.float32),
        mesh=mesh,
        compiler_params=pltpu.CompilerParams(needs_layout_passes=False),
        scratch_types=[
            pltpu.VMEM((m,), jnp.int32),
            pltpu.VMEM((m + 32,), jnp.int32),
            pltpu.VMEM((n,), jnp.float32),
        ],
    )
    def sc_kernel(idx_hbm, seg_hbm, c_out, seg_v, idx_v, crow_v):
        c = lax.axis_index("c")
        t = lax.axis_index("s")
        pltpu.sync_copy(seg_hbm, seg_v)
        pltpu.sync_copy(idx_hbm, idx_v.at[pl.ds(0, m)])
        lanes = lax.iota(jnp.int32, 16)
        one = jnp.float32(1.0)
        nil = jnp.float32(0.0)
        zf16 = jnp.zeros((16,), jnp.float32)
        ones16 = jnp.full((16,), 1.0, jnp.float32)

        def cz_body(k, carry):
            s_acc, e_acc = carry
            v = seg_v[pl.ds(k * 16, 16)]
            crow_v[pl.ds(k * 16, 16)] = zf16
            s_acc = s_acc + jnp.where(v < t, one, nil)
            e_acc = e_acc + jnp.where(v <= t, one, nil)
            return s_acc, e_acc

        assert m == n and m % 16 == 0
        s_acc, e_acc = lax.fori_loop(0, m // 16, cz_body, (zf16, zf16), unroll=8)
        start = jnp.sum(s_acc).astype(jnp.int32)
        end = jnp.sum(e_acc).astype(jnp.int32)

        mid = (start + end) // 2
        h0 = jnp.where(c == 0, start, mid)
        h1 = jnp.where(c == 0, mid, end)
        nch = (h1 - h0 + 31) // 32

        def sc_body(i, carry):
            pos = h0 + i * 32
            valid0 = (lanes + pos) < h1
            valid1 = (lanes + (pos + 16)) < h1
            iv0 = idx_v[pl.ds(pos, 16)]
            iv1 = idx_v[pl.ds(pos + 16, 16)]
            plsc.addupdate_scatter(crow_v, [iv0], ones16, mask=valid0)
            plsc.addupdate_scatter(crow_v, [iv1], ones16, mask=valid1)
            return carry

        lax.fori_loop(0, nch, sc_body, 0)
        pltpu.sync_copy(crow_v, c_out.at[c, t])

    return sc_kernel


def _mv_body(z_ref, w_ref, c2_ref, y_ref, md_ref, m_run, d_run):
    k = pl.program_id(0)
    nseg = c2_ref.shape[1]
    blk = c2_ref.shape[2]
    y_blk = jnp.sum(z_ref[...] * w_ref[...], axis=1)
    y_ref[...] = y_blk[None, None, :]

    @pl.when(k == 0)
    def _():
        m_run[...] = jnp.full((nseg, 1), _NEG, jnp.float32)
        d_run[...] = jnp.zeros((nseg, 1), jnp.float32)

    cb = c2_ref[0] + c2_ref[1]
    yb = jnp.broadcast_to(y_ref[0], (nseg, blk))
    ymasked = jnp.where(cb > 0.0, yb, _NEG)
    bmax = jnp.max(ymasked, axis=1, keepdims=True)
    m_new = jnp.maximum(m_run[...], bmax)
    e_blk = jnp.where(cb > 0.0, jnp.exp(yb - m_new), 0.0)
    d_new = d_run[...] * jnp.exp(m_run[...] - m_new) + jnp.sum(
        cb * e_blk, axis=1, keepdims=True
    )
    m_run[...] = m_new
    d_run[...] = d_new

    @pl.when(k == pl.num_programs(0) - 1)
    def _():
        m_fin = jnp.where(m_new == _NEG, 0.0, m_new)
        d_fin = jnp.where(d_new == 0.0, 1.0, d_new)
        md_ref[:, 0:16] = jnp.broadcast_to(m_fin, (nseg, 16))
        md_ref[:, 16:32] = jnp.broadcast_to(d_fin, (nseg, 16))


def _matvec_stats(z, w, c2, num_segments):
    n, dim = z.shape
    blk = 2048
    grid = n // blk
    return pl.pallas_call(
        _mv_body,
        grid=(grid,),
        in_specs=[
            pl.BlockSpec((blk, dim), lambda k: (k, 0)),
            pl.BlockSpec((1, dim), lambda k: (0, 0)),
            pl.BlockSpec((2, num_segments, blk), lambda k: (0, 0, k)),
        ],
        out_specs=[
            pl.BlockSpec((1, 1, blk), lambda k: (k, 0, 0)),
            pl.BlockSpec((num_segments, 32), lambda k: (0, 0)),
        ],
        out_shape=[
            jax.ShapeDtypeStruct((grid, 1, blk), jnp.float32),
            jax.ShapeDtypeStruct((num_segments, 32), jnp.float32),
        ],
        scratch_shapes=[
            pltpu.VMEM((num_segments, 1), jnp.float32),
            pltpu.VMEM((num_segments, 1), jnp.float32),
        ],
    )(z, w.reshape(1, dim), c2)


def _mm_body(c2_ref, md_ref, y_ref, z_ref, out_ref):
    k = pl.program_id(0)
    nseg = c2_ref.shape[1]
    blk = c2_ref.shape[2]
    mvec = md_ref[:, 0:1]
    inv_d = 1.0 / md_ref[:, 16:17]
    cb = c2_ref[0] + c2_ref[1]
    yb = jnp.broadcast_to(y_ref[0], (nseg, blk))
    s_blk = jnp.where(cb > 0.0, cb * jnp.exp(yb - mvec) * inv_d, 0.0)
    part = jnp.dot(s_blk, z_ref[...], preferred_element_type=jnp.float32)

    @pl.when(k == 0)
    def _():
        out_ref[...] = part

    @pl.when(k > 0)
    def _():
        out_ref[...] += part


def _weighted_matmul(c2, md, y3d, z, num_segments):
    n, dim = z.shape
    grid, _, blk = y3d.shape
    return pl.pallas_call(
        _mm_body,
        grid=(grid,),
        in_specs=[
            pl.BlockSpec((2, num_segments, blk), lambda k: (0, 0, k)),
            pl.BlockSpec((num_segments, 32), lambda k: (0, 0)),
            pl.BlockSpec((1, 1, blk), lambda k: (k, 0, 0)),
            pl.BlockSpec((blk, dim), lambda k: (k, 0)),
        ],
        out_specs=pl.BlockSpec((num_segments, dim), lambda k: (0, 0)),
        out_shape=jax.ShapeDtypeStruct((num_segments, dim), jnp.float32),
    )(c2, md, y3d, z)


def kernel(z, w_attn, b_attn, flat_idx, segment_ids):
    del b_attn
    n, dim = z.shape
    (m,) = flat_idx.shape
    num_segments = 16
    idx32 = flat_idx.astype(jnp.int32)
    seg32 = segment_ids.astype(jnp.int32)
    c2 = _make_sc_counts(m, n, num_segments)(idx32, seg32)
    y3d, md = _matvec_stats(z, w_attn, c2, num_segments)
    return _weighted_matmul(c2, md, y3d, z, num_segments)


# speedup vs baseline: 1.4441x; 1.0375x over previous
import functools

import numpy as np

import jax
import jax.numpy as jnp
from jax import lax
from jax.experimental import pallas as pl
from jax.experimental.pallas import tpu as pltpu
from jax.experimental.pallas import tpu_sc as plsc

_NEG = np.float32(-3.0e38)


def _make_sc_counts(m, n, num_segments):
    mesh = plsc.VectorSubcoreMesh(core_axis_name="c", subcore_axis_name="s")

    @functools.partial(
        pl.kernel,
        out_type=jax.ShapeDtypeStruct((2, num_segments, n), jnp.float32),
        mesh=mesh,
        compiler_params=pltpu.CompilerParams(needs_layout_passes=False),
        scratch_types=[
            pltpu.VMEM((m,), jnp.int32),
            pltpu.VMEM((m + 32,), jnp.int32),
            pltpu.VMEM((n,), jnp.float32),
        ],
    )
    def sc_kernel(idx_hbm, seg_hbm, c_out, seg_v, idx_v, crow_v):
        c = lax.axis_index("c")
        t = lax.axis_index("s")
        pltpu.sync_copy(seg_hbm, seg_v)
        pltpu.sync_copy(idx_hbm, idx_v.at[pl.ds(0, m)])
        lanes = lax.iota(jnp.int32, 16)
        one = jnp.float32(1.0)
        nil = jnp.float32(0.0)
        zf16 = jnp.zeros((16,), jnp.float32)
        ones16 = jnp.full((16,), 1.0, jnp.float32)

        def cz_body(k, carry):
            s_acc, e_acc = carry
            v = seg_v[pl.ds(k * 16, 16)]
            crow_v[pl.ds(k * 16, 16)] = zf16
            s_acc = s_acc + jnp.where(v < t, one, nil)
            e_acc = e_acc + jnp.where(v <= t, one, nil)
            return s_acc, e_acc

        assert m == n and m % 16 == 0
        s_acc, e_acc = lax.fori_loop(0, m // 16, cz_body, (zf16, zf16), unroll=8)
        start = jnp.sum(s_acc).astype(jnp.int32)
        end = jnp.sum(e_acc).astype(jnp.int32)

        mid = (start + end) // 2
        h0 = jnp.where(c == 0, start, mid)
        h1 = jnp.where(c == 0, mid, end)
        nch = (h1 - h0 + 31) // 32

        def sc_body(i, carry):
            pos = h0 + i * 32
            valid0 = (lanes + pos) < h1
            valid1 = (lanes + (pos + 16)) < h1
            iv0 = idx_v[pl.ds(pos, 16)]
            iv1 = idx_v[pl.ds(pos + 16, 16)]
            plsc.addupdate_scatter(crow_v, [iv0], ones16, mask=valid0)
            plsc.addupdate_scatter(crow_v, [iv1], ones16, mask=valid1)
            return carry

        lax.fori_loop(0, nch, sc_body, 0)
        pltpu.sync_copy(crow_v, c_out.at[c, t])

    return sc_kernel


def _fused_body(z_ref, w_ref, c2_ref, out_ref, y_s, m_run, d_run):
    p = pl.program_id(0)
    k = pl.program_id(1)
    nseg = c2_ref.shape[1]
    blk = c2_ref.shape[2]
    cb = c2_ref[0] + c2_ref[1]

    @pl.when(p == 0)
    def _():
        y_blk = jnp.sum(z_ref[...] * w_ref[...], axis=1)
        y_s[pl.ds(k, 1)] = y_blk[None, None, :]

        @pl.when(k == 0)
        def _():
            m_run[...] = jnp.full((nseg, 1), _NEG, jnp.float32)
            d_run[...] = jnp.zeros((nseg, 1), jnp.float32)

        yb = jnp.broadcast_to(y_s[pl.ds(k, 1)][0], (nseg, blk))
        ymasked = jnp.where(cb > 0.0, yb, _NEG)
        bmax = jnp.max(ymasked, axis=1, keepdims=True)
        m_new = jnp.maximum(m_run[...], bmax)
        e_blk = jnp.where(cb > 0.0, jnp.exp(yb - m_new), 0.0)
        d_new = d_run[...] * jnp.exp(m_run[...] - m_new) + jnp.sum(
            cb * e_blk, axis=1, keepdims=True
        )
        m_run[...] = m_new
        d_run[...] = d_new

        @pl.when(k == pl.num_programs(1) - 1)
        def _():
            m_run[...] = jnp.where(m_new == _NEG, 0.0, m_new)
            d_run[...] = jnp.where(d_new == 0.0, 1.0, d_new)

    @pl.when(p == 1)
    def _():
        mvec = m_run[...]
        inv_d = 1.0 / d_run[...]
        yb = jnp.broadcast_to(y_s[pl.ds(k, 1)][0], (nseg, blk))
        s_blk = jnp.where(cb > 0.0, cb * jnp.exp(yb - mvec) * inv_d, 0.0)
        part = jnp.dot(s_blk, z_ref[...], preferred_element_type=jnp.float32)

        @pl.when(k == 0)
        def _():
            out_ref[...] = part

        @pl.when(k > 0)
        def _():
            out_ref[...] += part


def _fused_tc(z, w, c2, num_segments):
    n, dim = z.shape
    blk = 2048
    grid = n // blk
    return pl.pallas_call(
        _fused_body,
        grid=(2, grid),
        in_specs=[
            pl.BlockSpec((blk, dim), lambda p, k: (k, 0)),
            pl.BlockSpec((1, dim), lambda p, k: (0, 0)),
            pl.BlockSpec((2, num_segments, blk), lambda p, k: (0, 0, k)),
        ],
        out_specs=pl.BlockSpec((num_segments, dim), lambda p, k: (0, 0)),
        out_shape=jax.ShapeDtypeStruct((num_segments, dim), jnp.float32),
        scratch_shapes=[
            pltpu.VMEM((grid, 1, blk), jnp.float32),
            pltpu.VMEM((num_segments, 1), jnp.float32),
            pltpu.VMEM((num_segments, 1), jnp.float32),
        ],
    )(z, w.reshape(1, dim), c2)


def kernel(z, w_attn, b_attn, flat_idx, segment_ids):
    del b_attn
    n, dim = z.shape
    (m,) = flat_idx.shape
    num_segments = 16
    idx32 = flat_idx.astype(jnp.int32)
    seg32 = segment_ids.astype(jnp.int32)
    c2 = _make_sc_counts(m, n, num_segments)(idx32, seg32)
    return _fused_tc(z, w_attn, c2, num_segments)


# speedup vs baseline: 1.4821x; 1.0264x over previous
import functools

import numpy as np

import jax
import jax.numpy as jnp
from jax import lax
from jax.experimental import pallas as pl
from jax.experimental.pallas import tpu as pltpu
from jax.experimental.pallas import tpu_sc as plsc

_NEG = np.float32(-3.0e38)


def _make_sc_counts(m, n, num_segments):
    mesh = plsc.VectorSubcoreMesh(core_axis_name="c", subcore_axis_name="s")

    @functools.partial(
        pl.kernel,
        out_type=jax.ShapeDtypeStruct((2, num_segments, n), jnp.float32),
        mesh=mesh,
        compiler_params=pltpu.CompilerParams(needs_layout_passes=False),
        scratch_types=[
            pltpu.VMEM((m,), jnp.int32),
            pltpu.VMEM((m + 32,), jnp.int32),
            pltpu.VMEM((n,), jnp.float32),
        ],
    )
    def sc_kernel(idx_hbm, seg_hbm, c_out, seg_v, idx_v, crow_v):
        c = lax.axis_index("c")
        t = lax.axis_index("s")
        pltpu.sync_copy(seg_hbm, seg_v)
        pltpu.sync_copy(idx_hbm, idx_v.at[pl.ds(0, m)])
        lanes = lax.iota(jnp.int32, 16)
        one = jnp.float32(1.0)
        nil = jnp.float32(0.0)
        zf16 = jnp.zeros((16,), jnp.float32)
        ones16 = jnp.full((16,), 1.0, jnp.float32)

        def cz_body(k, carry):
            s_acc, e_acc = carry
            v = seg_v[pl.ds(k * 16, 16)]
            crow_v[pl.ds(k * 16, 16)] = zf16
            s_acc = s_acc + jnp.where(v < t, one, nil)
            e_acc = e_acc + jnp.where(v <= t, one, nil)
            return s_acc, e_acc

        assert m == n and m % 16 == 0
        s_acc, e_acc = lax.fori_loop(0, m // 16, cz_body, (zf16, zf16), unroll=8)
        start = jnp.sum(s_acc).astype(jnp.int32)
        end = jnp.sum(e_acc).astype(jnp.int32)

        mid = (start + end) // 2
        h0 = jnp.where(c == 0, start, mid)
        h1 = jnp.where(c == 0, mid, end)
        nch = (h1 - h0 + 31) // 32

        def sc_body(i, carry):
            pos = h0 + i * 32
            valid0 = (lanes + pos) < h1
            valid1 = (lanes + (pos + 16)) < h1
            iv0 = idx_v[pl.ds(pos, 16)]
            iv1 = idx_v[pl.ds(pos + 16, 16)]
            plsc.addupdate_scatter(crow_v, [iv0], ones16, mask=valid0)
            plsc.addupdate_scatter(crow_v, [iv1], ones16, mask=valid1)
            return carry

        lax.fori_loop(0, nch, sc_body, 0)
        pltpu.sync_copy(crow_v, c_out.at[c, t])

    return sc_kernel


def _fused_body(z_ref, w_ref, c2_ref, out_ref, y_s, m_run, d_run):
    p = pl.program_id(0)
    k = pl.program_id(1)
    nseg = c2_ref.shape[1]
    blk = c2_ref.shape[2]
    cb = c2_ref[0] + c2_ref[1]

    @pl.when(p == 0)
    def _():
        y_blk2d = lax.dot_general(
            w_ref[...], z_ref[...], (((1,), (1,)), ((), ())),
            preferred_element_type=jnp.float32,
        )
        y_s[pl.ds(k, 1)] = y_blk2d[None]

        @pl.when(k == 0)
        def _():
            m_run[...] = jnp.full((nseg, 1), _NEG, jnp.float32)
            d_run[...] = jnp.zeros((nseg, 1), jnp.float32)

        yb = jnp.broadcast_to(y_s[pl.ds(k, 1)][0], (nseg, blk))
        ymasked = jnp.where(cb > 0.0, yb, _NEG)
        bmax = jnp.max(ymasked, axis=1, keepdims=True)
        m_new = jnp.maximum(m_run[...], bmax)
        e_blk = jnp.where(cb > 0.0, jnp.exp(yb - m_new), 0.0)
        d_new = d_run[...] * jnp.exp(m_run[...] - m_new) + jnp.sum(
            cb * e_blk, axis=1, keepdims=True
        )
        m_run[...] = m_new
        d_run[...] = d_new

        @pl.when(k == pl.num_programs(1) - 1)
        def _():
            m_run[...] = jnp.where(m_new == _NEG, 0.0, m_new)
            d_run[...] = jnp.where(d_new == 0.0, 1.0, d_new)

    @pl.when(p == 1)
    def _():
        mvec = m_run[...]
        inv_d = 1.0 / d_run[...]
        yb = jnp.broadcast_to(y_s[pl.ds(k, 1)][0], (nseg, blk))
        s_blk = jnp.where(cb > 0.0, cb * jnp.exp(yb - mvec) * inv_d, 0.0)
        part = jnp.dot(s_blk, z_ref[...], preferred_element_type=jnp.float32)

        @pl.when(k == 0)
        def _():
            out_ref[...] = part

        @pl.when(k > 0)
        def _():
            out_ref[...] += part


def _fused_tc(z, w, c2, num_segments):
    n, dim = z.shape
    blk = 2048
    grid = n // blk
    return pl.pallas_call(
        _fused_body,
        grid=(2, grid),
        in_specs=[
            pl.BlockSpec((blk, dim), lambda p, k: (k, 0)),
            pl.BlockSpec((1, dim), lambda p, k: (0, 0)),
            pl.BlockSpec((2, num_segments, blk), lambda p, k: (0, 0, k)),
        ],
        out_specs=pl.BlockSpec((num_segments, dim), lambda p, k: (0, 0)),
        out_shape=jax.ShapeDtypeStruct((num_segments, dim), jnp.float32),
        scratch_shapes=[
            pltpu.VMEM((grid, 1, blk), jnp.float32),
            pltpu.VMEM((num_segments, 1), jnp.float32),
            pltpu.VMEM((num_segments, 1), jnp.float32),
        ],
    )(z, w.reshape(1, dim), c2)


def kernel(z, w_attn, b_attn, flat_idx, segment_ids):
    del b_attn
    n, dim = z.shape
    (m,) = flat_idx.shape
    num_segments = 16
    idx32 = flat_idx.astype(jnp.int32)
    seg32 = segment_ids.astype(jnp.int32)
    c2 = _make_sc_counts(m, n, num_segments)(idx32, seg32)
    return _fused_tc(z, w_attn, c2, num_segments)


# speedup vs baseline: 1.9350x; 1.3055x over previous
import functools

import numpy as np

import jax
import jax.numpy as jnp
from jax import lax
from jax.experimental import pallas as pl
from jax.experimental.pallas import tpu as pltpu
from jax.experimental.pallas import tpu_sc as plsc

_NEG = np.float32(-3.0e38)


def _make_sc_counts(m, n, num_segments):
    mesh = plsc.VectorSubcoreMesh(core_axis_name="c", subcore_axis_name="s")

    @functools.partial(
        pl.kernel,
        out_type=jax.ShapeDtypeStruct((2, num_segments, n), jnp.float32),
        mesh=mesh,
        compiler_params=pltpu.CompilerParams(needs_layout_passes=False),
        scratch_types=[
            pltpu.VMEM((m,), jnp.int32),
            pltpu.VMEM((m + 32,), jnp.int32),
            pltpu.VMEM((n,), jnp.float32),
        ],
    )
    def sc_kernel(idx_hbm, seg_hbm, c_out, seg_v, idx_v, crow_v):
        c = lax.axis_index("c")
        t = lax.axis_index("s")
        pltpu.sync_copy(seg_hbm, seg_v)
        pltpu.sync_copy(idx_hbm, idx_v.at[pl.ds(0, m)])
        lanes = lax.iota(jnp.int32, 16)
        one = jnp.float32(1.0)
        nil = jnp.float32(0.0)
        zf16 = jnp.zeros((16,), jnp.float32)
        ones16 = jnp.full((16,), 1.0, jnp.float32)

        def cz_body(k, carry):
            s_acc, e_acc = carry
            v = seg_v[pl.ds(k * 16, 16)]
            crow_v[pl.ds(k * 16, 16)] = zf16
            s_acc = s_acc + jnp.where(v < t, one, nil)
            e_acc = e_acc + jnp.where(v <= t, one, nil)
            return s_acc, e_acc

        assert m == n and m % 16 == 0
        s_acc, e_acc = lax.fori_loop(0, m // 16, cz_body, (zf16, zf16), unroll=8)
        start = jnp.sum(s_acc).astype(jnp.int32)
        end = jnp.sum(e_acc).astype(jnp.int32)

        mid = (start + end) // 2
        h0 = jnp.where(c == 0, start, mid)
        h1 = jnp.where(c == 0, mid, end)
        nch = (h1 - h0 + 31) // 32

        def sc_body(i, carry):
            pos = h0 + i * 32
            valid0 = (lanes + pos) < h1
            valid1 = (lanes + (pos + 16)) < h1
            iv0 = idx_v[pl.ds(pos, 16)]
            iv1 = idx_v[pl.ds(pos + 16, 16)]
            plsc.addupdate_scatter(crow_v, [iv0], ones16, mask=valid0)
            plsc.addupdate_scatter(crow_v, [iv1], ones16, mask=valid1)
            return carry

        lax.fori_loop(0, nch, sc_body, 0)
        pltpu.sync_copy(crow_v, c_out.at[c, t])

    return sc_kernel


def _fused_body(z_ref, w_ref, c2_ref, out_ref, out_run, m_run, d_run):
    k = pl.program_id(0)
    nseg = c2_ref.shape[1]
    blk = c2_ref.shape[2]
    cb = c2_ref[0] + c2_ref[1]

    @pl.when(k == 0)
    def _():
        m_run[...] = jnp.full((nseg, 1), _NEG, jnp.float32)
        d_run[...] = jnp.zeros((nseg, 1), jnp.float32)
        out_run[...] = jnp.zeros_like(out_run)

    y_blk = lax.dot_general(
        w_ref[...], z_ref[...], (((1,), (1,)), ((), ())),
        preferred_element_type=jnp.float32,
    )
    yb = jnp.broadcast_to(y_blk, (nseg, blk))
    ymasked = jnp.where(cb > 0.0, yb, _NEG)
    bmax = jnp.max(ymasked, axis=1, keepdims=True)
    m_new = jnp.maximum(m_run[...], bmax)
    scale = jnp.exp(m_run[...] - m_new)
    e_blk = jnp.where(cb > 0.0, cb * jnp.exp(yb - m_new), 0.0)
    d_run[...] = d_run[...] * scale + jnp.sum(e_blk, axis=1, keepdims=True)
    part = jnp.dot(e_blk, z_ref[...], preferred_element_type=jnp.float32)
    out_run[...] = out_run[...] * scale + part
    m_run[...] = m_new

    @pl.when(k == pl.num_programs(0) - 1)
    def _():
        d_fin = jnp.where(d_run[...] == 0.0, 1.0, d_run[...])
        out_ref[...] = out_run[...] / d_fin


def _fused_tc(z, w, c2, num_segments):
    n, dim = z.shape
    blk = 2048
    grid = n // blk
    return pl.pallas_call(
        _fused_body,
        grid=(grid,),
        in_specs=[
            pl.BlockSpec((blk, dim), lambda k: (k, 0)),
            pl.BlockSpec((1, dim), lambda k: (0, 0)),
            pl.BlockSpec((2, num_segments, blk), lambda k: (0, 0, k)),
        ],
        out_specs=pl.BlockSpec((num_segments, dim), lambda k: (0, 0)),
        out_shape=jax.ShapeDtypeStruct((num_segments, dim), jnp.float32),
        scratch_shapes=[
            pltpu.VMEM((num_segments, dim), jnp.float32),
            pltpu.VMEM((num_segments, 1), jnp.float32),
            pltpu.VMEM((num_segments, 1), jnp.float32),
        ],
    )(z, w.reshape(1, dim), c2)


def kernel(z, w_attn, b_attn, flat_idx, segment_ids):
    del b_attn
    n, dim = z.shape
    (m,) = flat_idx.shape
    num_segments = 16
    idx32 = flat_idx.astype(jnp.int32)
    seg32 = segment_ids.astype(jnp.int32)
    c2 = _make_sc_counts(m, n, num_segments)(idx32, seg32)
    return _fused_tc(z, w_attn, c2, num_segments)


# speedup vs baseline: 1.9355x; 1.0003x over previous
import functools

import numpy as np

import jax
import jax.numpy as jnp
from jax import lax
from jax.experimental import pallas as pl
from jax.experimental.pallas import tpu as pltpu
from jax.experimental.pallas import tpu_sc as plsc

_NEG = np.float32(-3.0e38)


def _make_sc_counts(m, n, num_segments):
    mesh = plsc.VectorSubcoreMesh(core_axis_name="c", subcore_axis_name="s")

    @functools.partial(
        pl.kernel,
        out_type=jax.ShapeDtypeStruct((2, num_segments, n), jnp.float32),
        mesh=mesh,
        compiler_params=pltpu.CompilerParams(needs_layout_passes=False),
        scratch_types=[
            pltpu.VMEM((m,), jnp.int32),
            pltpu.VMEM((m + 32,), jnp.int32),
            pltpu.VMEM((n,), jnp.float32),
        ],
    )
    def sc_kernel(idx_hbm, seg_hbm, c_out, seg_v, idx_v, crow_v):
        c = lax.axis_index("c")
        t = lax.axis_index("s")
        pltpu.sync_copy(seg_hbm, seg_v)
        pltpu.sync_copy(idx_hbm, idx_v.at[pl.ds(0, m)])
        lanes = lax.iota(jnp.int32, 16)
        one = jnp.float32(1.0)
        nil = jnp.float32(0.0)
        zf16 = jnp.zeros((16,), jnp.float32)
        ones16 = jnp.full((16,), 1.0, jnp.float32)

        def cz_body(k, carry):
            s_acc, e_acc = carry
            v = seg_v[pl.ds(k * 16, 16)]
            crow_v[pl.ds(k * 16, 16)] = zf16
            s_acc = s_acc + jnp.where(v < t, one, nil)
            e_acc = e_acc + jnp.where(v <= t, one, nil)
            return s_acc, e_acc

        assert m == n and m % 16 == 0
        s_acc, e_acc = lax.fori_loop(0, m // 16, cz_body, (zf16, zf16), unroll=8)
        start = jnp.sum(s_acc).astype(jnp.int32)
        end = jnp.sum(e_acc).astype(jnp.int32)

        mid = (start + end) // 2
        h0 = jnp.where(c == 0, start, mid)
        h1 = jnp.where(c == 0, mid, end)
        nch = (h1 - h0 + 31) // 32

        def sc_body(i, carry):
            pos = h0 + i * 32
            valid0 = (lanes + pos) < h1
            valid1 = (lanes + (pos + 16)) < h1
            iv0 = idx_v[pl.ds(pos, 16)]
            iv1 = idx_v[pl.ds(pos + 16, 16)]
            plsc.addupdate_scatter(crow_v, [iv0], ones16, mask=valid0)
            plsc.addupdate_scatter(crow_v, [iv1], ones16, mask=valid1)
            return carry

        lax.fori_loop(0, nch, sc_body, 0)
        pltpu.sync_copy(crow_v, c_out.at[c, t])

    return sc_kernel


def _fused_body(z_ref, w_ref, c2_ref, out_ref, out_run, m_run, d_run):
    k = pl.program_id(0)
    nseg = c2_ref.shape[1]
    blk = c2_ref.shape[2]
    cb = c2_ref[0] + c2_ref[1]

    @pl.when(k == 0)
    def _():
        m_run[...] = jnp.full((nseg, 1), _NEG, jnp.float32)
        d_run[...] = jnp.zeros((nseg, 1), jnp.float32)
        out_run[...] = jnp.zeros_like(out_run)

    y_blk = lax.dot_general(
        w_ref[...], z_ref[...], (((1,), (1,)), ((), ())),
        preferred_element_type=jnp.float32,
    )
    yb = jnp.broadcast_to(y_blk, (nseg, blk))
    ymasked = jnp.where(cb > 0.0, yb, _NEG)
    bmax = jnp.max(ymasked, axis=1, keepdims=True)
    m_new = jnp.maximum(m_run[...], bmax)
    scale = jnp.exp(m_run[...] - m_new)
    e_blk = jnp.where(cb > 0.0, cb * jnp.exp(yb - m_new), 0.0)
    d_run[...] = d_run[...] * scale + jnp.sum(e_blk, axis=1, keepdims=True)
    part = jnp.dot(
        e_blk.astype(jnp.bfloat16),
        z_ref[...].astype(jnp.bfloat16),
        preferred_element_type=jnp.float32,
    )
    out_run[...] = out_run[...] * scale + part
    m_run[...] = m_new

    @pl.when(k == pl.num_programs(0) - 1)
    def _():
        d_fin = jnp.where(d_run[...] == 0.0, 1.0, d_run[...])
        out_ref[...] = out_run[...] / d_fin


def _fused_tc(z, w, c2, num_segments):
    n, dim = z.shape
    blk = 2048
    grid = n // blk
    return pl.pallas_call(
        _fused_body,
        grid=(grid,),
        in_specs=[
            pl.BlockSpec((blk, dim), lambda k: (k, 0)),
            pl.BlockSpec((1, dim), lambda k: (0, 0)),
            pl.BlockSpec((2, num_segments, blk), lambda k: (0, 0, k)),
        ],
        out_specs=pl.BlockSpec((num_segments, dim), lambda k: (0, 0)),
        out_shape=jax.ShapeDtypeStruct((num_segments, dim), jnp.float32),
        scratch_shapes=[
            pltpu.VMEM((num_segments, dim), jnp.float32),
            pltpu.VMEM((num_segments, 1), jnp.float32),
            pltpu.VMEM((num_segments, 1), jnp.float32),
        ],
    )(z, w.reshape(1, dim), c2)


def kernel(z, w_attn, b_attn, flat_idx, segment_ids):
    del b_attn
    n, dim = z.shape
    (m,) = flat_idx.shape
    num_segments = 16
    idx32 = flat_idx.astype(jnp.int32)
    seg32 = segment_ids.astype(jnp.int32)
    c2 = _make_sc_counts(m, n, num_segments)(idx32, seg32)
    return _fused_tc(z, w_attn, c2, num_segments)
